# baseline (speedup 1.0000x reference)
import functools

import jax
import jax.numpy as jnp
from jax import lax
from jax.experimental import pallas as pl
from jax.experimental.pallas import tpu as pltpu
from jax.experimental.pallas import tpu_sc as plsc


def _sc_prep_body(V, VR, DP, E, EP, ZV,
                  ei_hbm, m_hbm, deg_hbm,
                  mv, degv, srcv, dstv):
    c = lax.axis_index("c")
    s = lax.axis_index("s")

    @pl.when(jnp.logical_and(c == 0, s == 0))
    def _():
        pltpu.sync_copy(ei_hbm.at[pl.ds(0, EP)], srcv)
        pltpu.sync_copy(ei_hbm.at[pl.ds(EP, EP)], dstv)
        zv16 = jnp.zeros((16,), jnp.float32)

        def zero_body(i, carry):
            base = i * 128
            for k in range(8):
                mv[pl.ds(base + k * 16, 16)] = zv16
            return carry

        lax.fori_loop(0, ZV // 128, zero_body, 0)

        def zero_deg(i, carry):
            degv[pl.ds(i * 16, 16)] = zv16
            return carry

        lax.fori_loop(0, DP // 16, zero_deg, 0)

        ones = jnp.ones((16,), jnp.float32)
        lane = lax.iota(jnp.int32, 16)

        def edge_body(i, carry):
            base = i * 16
            mask = (base + lane) < E
            sv = jnp.where(mask, srcv[pl.ds(base, 16)], 0)
            dv = jnp.where(mask, dstv[pl.ds(base, 16)], 0)
            idx = dv * VR + sv
            plsc.addupdate_scatter(mv, [idx], ones, mask=mask)
            plsc.addupdate_scatter(degv, [dv], ones, mask=mask)
            return carry

        lax.fori_loop(0, (E + 15) // 16, edge_body, 0)

        def diag_body(i, carry):
            base = i * 16
            v = base + lane
            mask = v < V
            idx = v * (VR + 1)
            plsc.addupdate_scatter(mv, [idx], ones, mask=mask)
            cur = degv[pl.ds(base, 16)]
            degv[pl.ds(base, 16)] = cur + jnp.where(mask, 1.0, 0.0)
            return carry

        lax.fori_loop(0, DP // 16, diag_body, 0)

        pltpu.sync_copy(mv, m_hbm)
        pltpu.sync_copy(degv.at[pl.ds(0, VR)], deg_hbm)


def _sc_prep(ei_flat, V, VR, E, EP, ZV):
    mesh = plsc.VectorSubcoreMesh(core_axis_name="c", subcore_axis_name="s")
    DP = VR
    body = functools.partial(_sc_prep_body, V, VR, DP, E, EP, ZV)
    return pl.kernel(
        body,
        out_type=(
            jax.ShapeDtypeStruct((ZV,), jnp.float32),
            jax.ShapeDtypeStruct((VR,), jnp.float32),
        ),
        mesh=mesh,
        compiler_params=pltpu.CompilerParams(needs_layout_passes=False),
        scratch_types=[
            pltpu.VMEM((ZV,), jnp.float32),
            pltpu.VMEM((DP,), jnp.float32),
            pltpu.VMEM((EP,), jnp.int32),
            pltpu.VMEM((EP,), jnp.int32),
        ],
    )(ei_flat)


def _fused_body(nb, N, V, C,
                x_r, w_r, b_r, m_r, degc_r, degr_r, gamma_r, beta_r,
                out_r, a_s, y_s, s1_r, s2_r, sc_r, sh_r):
    t = pl.program_id(0)
    j = pl.program_id(1)

    @pl.when(jnp.logical_and(t == 0, j == 0))
    def _():
        dinv_c = lax.rsqrt(degc_r[...])[:V, :]
        dinv_r = lax.rsqrt(degr_r[...])[:, :V]
        a_s[...] = m_r[:, :V] * dinv_c * dinv_r
        s1_r[...] = jnp.zeros_like(s1_r)
        s2_r[...] = jnp.zeros_like(s2_r)

    @pl.when(t == 0)
    def _():
        wv = w_r[...]
        bv = b_r[...]
        a16 = a_s[...].astype(jnp.bfloat16)
        s1 = jnp.zeros((V, C), jnp.float32)
        s2 = jnp.zeros((V, C), jnp.float32)
        base = j * nb
        for g in range(nb):
            xg = x_r[g]
            h = jnp.dot(xg, wv, preferred_element_type=jnp.float32) + bv
            hi = h.astype(jnp.bfloat16)
            lo = (h - hi.astype(jnp.float32)).astype(jnp.bfloat16)
            agg = (jnp.dot(a16, hi, preferred_element_type=jnp.float32)
                   + jnp.dot(a16, lo, preferred_element_type=jnp.float32))
            y = jnp.maximum(agg, 0.0)
            y_s[base + g] = y
            s1 = s1 + y
            s2 = s2 + y * y
        s1_r[...] = s1_r[...] + s1
        s2_r[...] = s2_r[...] + s2

    @pl.when(jnp.logical_and(t == 1, j == 0))
    def _():
        inv_n = 1.0 / N
        mean = s1_r[...] * inv_n
        var = s2_r[...] * inv_n - mean * mean
        rstd = lax.rsqrt(var + 1e-5)
        scale = rstd * gamma_r[...]
        sc_r[...] = scale
        sh_r[...] = beta_r[...] - mean * scale

    @pl.when(t == 1)
    def _():
        base = pl.multiple_of(j * nb, nb)
        yb = y_s[pl.ds(base, nb)]
        res = yb * sc_r[...][None, :, :] + sh_r[...][None, :, :]
        out_r[...] = res.reshape(nb, V * C)


def kernel(x, edge_index, W, b, gamma, beta):
    B_, T_, V, C = x.shape
    N = B_ * T_
    Co = W.shape[1]
    E = edge_index.shape[1]

    VR = ((V + 127) // 128) * 128
    ZV = V * VR
    EP = ((E + 127) // 128) * 128

    ei = jnp.pad(edge_index.astype(jnp.int32), ((0, 0), (0, EP - E))).reshape(-1)

    m_flat, deg_p = _sc_prep(ei, V, VR, E, EP, ZV)
    m = m_flat.reshape(V, VR)
    deg_c = deg_p.reshape(VR, 1)
    deg_r = deg_p.reshape(1, VR)

    x3 = x.reshape(N, V, C)
    b2 = b.reshape(1, Co)
    gamma2 = gamma.reshape(V, Co)
    beta2 = beta.reshape(V, Co)

    nb = 16
    NB = N // nb

    out = pl.pallas_call(
        functools.partial(_fused_body, nb, N, V, Co),
        grid=(2, NB),
        in_specs=[
            pl.BlockSpec((nb, V, C), lambda t, j: (j * (1 - t), 0, 0)),
            pl.BlockSpec((C, Co), lambda t, j: (0, 0)),
            pl.BlockSpec((1, Co), lambda t, j: (0, 0)),
            pl.BlockSpec((V, VR), lambda t, j: (0, 0)),
            pl.BlockSpec((VR, 1), lambda t, j: (0, 0)),
            pl.BlockSpec((1, VR), lambda t, j: (0, 0)),
            pl.BlockSpec((V, Co), lambda t, j: (0, 0)),
            pl.BlockSpec((V, Co), lambda t, j: (0, 0)),
        ],
        out_specs=pl.BlockSpec((nb, V * Co), lambda t, j: (j * t, 0)),
        out_shape=jax.ShapeDtypeStruct((N, V * Co), jnp.float32),
        scratch_shapes=[
            pltpu.VMEM((V, V), jnp.float32),
            pltpu.VMEM((N, V, Co), jnp.float32),
            pltpu.VMEM((V, Co), jnp.float32),
            pltpu.VMEM((V, Co), jnp.float32),
            pltpu.VMEM((V, Co), jnp.float32),
            pltpu.VMEM((V, Co), jnp.float32),
        ],
        compiler_params=pltpu.CompilerParams(
            dimension_semantics=("arbitrary", "arbitrary"),
        ),
    )(x3, W, b2, m, deg_c, deg_r, gamma2, beta2)

    return out.reshape(B_, T_, V * Co)


# speedup vs baseline: 1.1156x; 1.1156x over previous
import functools

import jax
import jax.numpy as jnp
from jax import lax
from jax.experimental import pallas as pl
from jax.experimental.pallas import tpu as pltpu
from jax.experimental.pallas import tpu_sc as plsc


def _sc_prep_body(V, VR, DP, E, EP, ZV,
                  ei_hbm, m_hbm, deg_hbm,
                  mv, degv, srcv, dstv):
    c = lax.axis_index("c")
    s = lax.axis_index("s")

    @pl.when(jnp.logical_and(c == 0, s == 0))
    def _():
        pltpu.sync_copy(ei_hbm.at[pl.ds(0, EP)], srcv)
        pltpu.sync_copy(ei_hbm.at[pl.ds(EP, EP)], dstv)
        zv16 = jnp.zeros((16,), jnp.float32)

        def zero_body(i, carry):
            base = i * 128
            for k in range(8):
                mv[pl.ds(base + k * 16, 16)] = zv16
            return carry

        lax.fori_loop(0, ZV // 128, zero_body, 0)

        def zero_deg(i, carry):
            degv[pl.ds(i * 16, 16)] = zv16
            return carry

        lax.fori_loop(0, DP // 16, zero_deg, 0)

        ones = jnp.ones((16,), jnp.float32)
        lane = lax.iota(jnp.int32, 16)

        def edge_body(i, carry):
            base = i * 16
            mask = (base + lane) < E
            sv = jnp.where(mask, srcv[pl.ds(base, 16)], 0)
            dv = jnp.where(mask, dstv[pl.ds(base, 16)], 0)
            idx = dv * VR + sv
            plsc.addupdate_scatter(mv, [idx], ones, mask=mask)
            plsc.addupdate_scatter(degv, [dv], ones, mask=mask)
            return carry

        lax.fori_loop(0, (E + 15) // 16, edge_body, 0)

        def diag_body(i, carry):
            base = i * 16
            v = base + lane
            mask = v < V
            idx = v * (VR + 1)
            plsc.addupdate_scatter(mv, [idx], ones, mask=mask)
            cur = degv[pl.ds(base, 16)]
            degv[pl.ds(base, 16)] = cur + jnp.where(mask, 1.0, 0.0)
            return carry

        lax.fori_loop(0, DP // 16, diag_body, 0)

        pltpu.sync_copy(mv, m_hbm)
        pltpu.sync_copy(degv.at[pl.ds(0, VR)], deg_hbm)


def _sc_prep(ei_flat, V, VR, E, EP, ZV):
    mesh = plsc.VectorSubcoreMesh(core_axis_name="c", subcore_axis_name="s")
    DP = VR
    body = functools.partial(_sc_prep_body, V, VR, DP, E, EP, ZV)
    return pl.kernel(
        body,
        out_type=(
            jax.ShapeDtypeStruct((ZV,), jnp.float32),
            jax.ShapeDtypeStruct((VR,), jnp.float32),
        ),
        mesh=mesh,
        compiler_params=pltpu.CompilerParams(needs_layout_passes=False),
        scratch_types=[
            pltpu.VMEM((ZV,), jnp.float32),
            pltpu.VMEM((DP,), jnp.float32),
            pltpu.VMEM((EP,), jnp.int32),
            pltpu.VMEM((EP,), jnp.int32),
        ],
    )(ei_flat)


def _fused_body(nb, N, V, C,
                x_r, w_r, b_r, m_r, degc_r, degr_r, gamma_r, beta_r,
                out_r, a_s, y_s, s1_r, s2_r, sc_r, sh_r):
    t = pl.program_id(0)
    j = pl.program_id(1)

    @pl.when(jnp.logical_and(t == 0, j == 0))
    def _():
        dinv_c = lax.rsqrt(degc_r[...])[:V, :]
        dinv_r = lax.rsqrt(degr_r[...])[:, :V]
        a_s[...] = m_r[:, :V] * dinv_c * dinv_r
        s1_r[...] = jnp.zeros_like(s1_r)
        s2_r[...] = jnp.zeros_like(s2_r)

    @pl.when(t == 0)
    def _():
        wv = w_r[...]
        bv = b_r[...]
        a16 = a_s[...].astype(jnp.bfloat16)
        s1 = jnp.zeros((V, C), jnp.float32)
        s2 = jnp.zeros((V, C), jnp.float32)
        base = j * nb
        for g in range(nb):
            xg = x_r[g]
            h = jnp.dot(xg, wv, preferred_element_type=jnp.float32) + bv
            agg = jnp.dot(a16, h.astype(jnp.bfloat16),
                          preferred_element_type=jnp.float32)
            y = jnp.maximum(agg, 0.0)
            y_s[base + g] = y
            s1 = s1 + y
            s2 = s2 + y * y
        s1_r[...] = s1_r[...] + s1
        s2_r[...] = s2_r[...] + s2

    @pl.when(jnp.logical_and(t == 1, j == 0))
    def _():
        inv_n = 1.0 / N
        mean = s1_r[...] * inv_n
        var = s2_r[...] * inv_n - mean * mean
        rstd = lax.rsqrt(var + 1e-5)
        scale = rstd * gamma_r[...]
        sc_r[...] = scale
        sh_r[...] = beta_r[...] - mean * scale

    @pl.when(t == 1)
    def _():
        base = pl.multiple_of(j * nb, nb)
        yb = y_s[pl.ds(base, nb)]
        res = yb * sc_r[...][None, :, :] + sh_r[...][None, :, :]
        out_r[...] = res.reshape(nb, V * C)


def kernel(x, edge_index, W, b, gamma, beta):
    B_, T_, V, C = x.shape
    N = B_ * T_
    Co = W.shape[1]
    E = edge_index.shape[1]

    VR = ((V + 127) // 128) * 128
    ZV = V * VR
    EP = ((E + 127) // 128) * 128

    ei = jnp.pad(edge_index.astype(jnp.int32), ((0, 0), (0, EP - E))).reshape(-1)

    m_flat, deg_p = _sc_prep(ei, V, VR, E, EP, ZV)
    m = m_flat.reshape(V, VR)
    deg_c = deg_p.reshape(VR, 1)
    deg_r = deg_p.reshape(1, VR)

    x3 = x.reshape(N, V, C)
    b2 = b.reshape(1, Co)
    gamma2 = gamma.reshape(V, Co)
    beta2 = beta.reshape(V, Co)

    nb = 16
    NB = N // nb

    out = pl.pallas_call(
        functools.partial(_fused_body, nb, N, V, Co),
        grid=(2, NB),
        in_specs=[
            pl.BlockSpec((nb, V, C), lambda t, j: (j * (1 - t), 0, 0)),
            pl.BlockSpec((C, Co), lambda t, j: (0, 0)),
            pl.BlockSpec((1, Co), lambda t, j: (0, 0)),
            pl.BlockSpec((V, VR), lambda t, j: (0, 0)),
            pl.BlockSpec((VR, 1), lambda t, j: (0, 0)),
            pl.BlockSpec((1, VR), lambda t, j: (0, 0)),
            pl.BlockSpec((V, Co), lambda t, j: (0, 0)),
            pl.BlockSpec((V, Co), lambda t, j: (0, 0)),
        ],
        out_specs=pl.BlockSpec((nb, V * Co), lambda t, j: (j * t, 0)),
        out_shape=jax.ShapeDtypeStruct((N, V * Co), jnp.float32),
        scratch_shapes=[
            pltpu.VMEM((V, V), jnp.float32),
            pltpu.VMEM((N, V, Co), jnp.float32),
            pltpu.VMEM((V, Co), jnp.float32),
            pltpu.VMEM((V, Co), jnp.float32),
            pltpu.VMEM((V, Co), jnp.float32),
            pltpu.VMEM((V, Co), jnp.float32),
        ],
        compiler_params=pltpu.CompilerParams(
            dimension_semantics=("arbitrary", "arbitrary"),
        ),
    )(x3, W, b2, m, deg_c, deg_r, gamma2, beta2)

    return out.reshape(B_, T_, V * Co)


# speedup vs baseline: 1.2061x; 1.0811x over previous
import functools

import jax
import jax.numpy as jnp
from jax import lax
from jax.experimental import pallas as pl
from jax.experimental.pallas import tpu as pltpu
from jax.experimental.pallas import tpu_sc as plsc


def _sc_prep_body(V, VR, E, EP, RP,
                  ei_hbm, m_hbm,
                  slab, srcv, dstv):
    c = lax.axis_index("c")
    s = lax.axis_index("s")
    nc = 2
    wid = s * nc + c
    lo = wid * RP

    pltpu.sync_copy(ei_hbm.at[pl.ds(0, EP)], srcv)
    pltpu.sync_copy(ei_hbm.at[pl.ds(EP, EP)], dstv)

    zv16 = jnp.zeros((16,), jnp.float32)

    def zero_body(i, carry):
        slab[pl.ds(i * 16, 16)] = zv16
        return carry

    lax.fori_loop(0, (RP * VR) // 16, zero_body, 0)

    ones = jnp.ones((16,), jnp.float32)
    lane = lax.iota(jnp.int32, 16)

    def edge_body(i, carry):
        base = i * 16
        mask = (base + lane) < E
        sv = jnp.where(mask, srcv[pl.ds(base, 16)], 0)
        dv = jnp.where(mask, dstv[pl.ds(base, 16)], 0)
        r = dv - lo
        mask = jnp.logical_and(mask,
                               jnp.logical_and(r >= 0, r < RP))
        idx = r * VR + sv
        plsc.addupdate_scatter(slab, [idx], ones, mask=mask)
        return carry

    lax.fori_loop(0, (E + 15) // 16, edge_body, 0)

    v = lo + lane
    dmask = jnp.logical_and(lane < RP, v < V)
    didx = lane * (VR + 1)
    plsc.addupdate_scatter(slab, [didx], ones, mask=dmask)

    pltpu.sync_copy(slab, m_hbm.at[pl.ds(lo * VR, RP * VR)])


def _sc_prep(ei_flat, V, VR, E, EP):
    mesh = plsc.VectorSubcoreMesh(core_axis_name="c", subcore_axis_name="s")
    RP = VR // 32
    body = functools.partial(_sc_prep_body, V, VR, E, EP, RP)
    return pl.kernel(
        body,
        out_type=jax.ShapeDtypeStruct((VR * VR,), jnp.float32),
        mesh=mesh,
        compiler_params=pltpu.CompilerParams(needs_layout_passes=False),
        scratch_types=[
            pltpu.VMEM((RP * VR,), jnp.float32),
            pltpu.VMEM((EP,), jnp.int32),
            pltpu.VMEM((EP,), jnp.int32),
        ],
    )(ei_flat)


def _fused_body(nb, N, V, C,
                x_r, w_r, b_r, m_r, gamma_r, beta_r,
                out_r, a_s, dv_s, y_s, s1_r, s2_r, sc_r, sh_r):
    t = pl.program_id(0)
    j = pl.program_id(1)

    @pl.when(jnp.logical_and(t == 0, j == 0))
    def _():
        msub = m_r[:V, :V]
        deg = jnp.sum(msub, axis=1, keepdims=True)
        dinv = lax.rsqrt(deg)
        dv_s[...] = dinv
        a_s[...] = msub * dinv
        s1_r[...] = jnp.zeros_like(s1_r)
        s2_r[...] = jnp.zeros_like(s2_r)

    @pl.when(t == 0)
    def _():
        wv = w_r[...]
        bv = b_r[...]
        a = a_s[...]
        dinv = dv_s[...]
        s1 = jnp.zeros((V, C), jnp.float32)
        s2 = jnp.zeros((V, C), jnp.float32)
        base = j * nb
        for g in range(nb):
            xg = x_r[g]
            h = jnp.dot(xg, wv, preferred_element_type=jnp.float32) + bv
            agg = jnp.dot(a, dinv * h, preferred_element_type=jnp.float32)
            y = jnp.maximum(agg, 0.0)
            y_s[base + g] = y
            s1 = s1 + y
            s2 = s2 + y * y
        s1_r[...] = s1_r[...] + s1
        s2_r[...] = s2_r[...] + s2

    @pl.when(jnp.logical_and(t == 1, j == 0))
    def _():
        inv_n = 1.0 / N
        mean = s1_r[...] * inv_n
        var = s2_r[...] * inv_n - mean * mean
        rstd = lax.rsqrt(var + 1e-5)
        scale = rstd * gamma_r[...]
        sc_r[...] = scale
        sh_r[...] = beta_r[...] - mean * scale

    @pl.when(t == 1)
    def _():
        base = pl.multiple_of(j * nb, nb)
        yb = y_s[pl.ds(base, nb)]
        res = yb * sc_r[...][None, :, :] + sh_r[...][None, :, :]
        out_r[...] = res.reshape(nb, V * C)


def kernel(x, edge_index, W, b, gamma, beta):
    B_, T_, V, C = x.shape
    N = B_ * T_
    Co = W.shape[1]
    E = edge_index.shape[1]

    VR = ((V + 127) // 128) * 128
    EP = ((E + 1023) // 1024) * 1024

    ei = jnp.pad(edge_index.astype(jnp.int32), ((0, 0), (0, EP - E))).reshape(-1)

    m_flat = _sc_prep(ei, V, VR, E, EP)
    m = m_flat.reshape(VR, VR)

    x3 = x.reshape(N, V, C)
    b2 = b.reshape(1, Co)
    gamma2 = gamma.reshape(V, Co)
    beta2 = beta.reshape(V, Co)

    nb = 16
    NB = N // nb

    out = pl.pallas_call(
        functools.partial(_fused_body, nb, N, V, Co),
        grid=(2, NB),
        in_specs=[
            pl.BlockSpec((nb, V, C), lambda t, j: (j * (1 - t), 0, 0)),
            pl.BlockSpec((C, Co), lambda t, j: (0, 0)),
            pl.BlockSpec((1, Co), lambda t, j: (0, 0)),
            pl.BlockSpec((VR, VR), lambda t, j: (0, 0)),
            pl.BlockSpec((V, Co), lambda t, j: (0, 0)),
            pl.BlockSpec((V, Co), lambda t, j: (0, 0)),
        ],
        out_specs=pl.BlockSpec((nb, V * Co), lambda t, j: (j * t, 0)),
        out_shape=jax.ShapeDtypeStruct((N, V * Co), jnp.float32),
        scratch_shapes=[
            pltpu.VMEM((V, V), jnp.float32),
            pltpu.VMEM((V, 1), jnp.float32),
            pltpu.VMEM((N, V, Co), jnp.float32),
            pltpu.VMEM((V, Co), jnp.float32),
            pltpu.VMEM((V, Co), jnp.float32),
            pltpu.VMEM((V, Co), jnp.float32),
            pltpu.VMEM((V, Co), jnp.float32),
        ],
        compiler_params=pltpu.CompilerParams(
            dimension_semantics=("arbitrary", "arbitrary"),
        ),
    )(x3, W, b2, m, gamma2, beta2)

    return out.reshape(B_, T_, V * Co)


# speedup vs baseline: 1.2090x; 1.0024x over previous
import functools

import jax
import jax.numpy as jnp
from jax import lax
from jax.experimental import pallas as pl
from jax.experimental.pallas import tpu as pltpu
from jax.experimental.pallas import tpu_sc as plsc


def _sc_prep_body(V, VR, E, EP, RP,
                  ei_hbm, m_hbm,
                  slab, srcv, dstv):
    c = lax.axis_index("c")
    s = lax.axis_index("s")
    nc = 2
    wid = s * nc + c
    lo = wid * RP

    pltpu.sync_copy(ei_hbm.at[pl.ds(0, EP)], srcv)
    pltpu.sync_copy(ei_hbm.at[pl.ds(EP, EP)], dstv)

    zv16 = jnp.zeros((16,), jnp.float32)

    def zero_body(i, carry):
        slab[pl.ds(i * 16, 16)] = zv16
        return carry

    lax.fori_loop(0, (RP * VR) // 16, zero_body, 0)

    ones = jnp.ones((16,), jnp.float32)
    lane = lax.iota(jnp.int32, 16)

    def edge_body(i, carry):
        base = i * 16
        mask = (base + lane) < E
        sv = jnp.where(mask, srcv[pl.ds(base, 16)], 0)
        dv = jnp.where(mask, dstv[pl.ds(base, 16)], 0)
        r = dv - lo
        mask = jnp.logical_and(mask,
                               jnp.logical_and(r >= 0, r < RP))
        idx = r * VR + sv
        plsc.addupdate_scatter(slab, [idx], ones, mask=mask)
        return carry

    lax.fori_loop(0, (E + 15) // 16, edge_body, 0)

    v = lo + lane
    dmask = jnp.logical_and(lane < RP, v < V)
    didx = lane * (VR + 1) + lo
    plsc.addupdate_scatter(slab, [didx], ones, mask=dmask)

    pltpu.sync_copy(slab, m_hbm.at[pl.ds(lo * VR, RP * VR)])


def _sc_prep(ei_flat, V, VR, E, EP):
    mesh = plsc.VectorSubcoreMesh(core_axis_name="c", subcore_axis_name="s")
    RP = VR // 32
    body = functools.partial(_sc_prep_body, V, VR, E, EP, RP)
    return pl.kernel(
        body,
        out_type=jax.ShapeDtypeStruct((VR * VR,), jnp.float32),
        mesh=mesh,
        compiler_params=pltpu.CompilerParams(needs_layout_passes=False),
        scratch_types=[
            pltpu.VMEM((RP * VR,), jnp.float32),
            pltpu.VMEM((EP,), jnp.int32),
            pltpu.VMEM((EP,), jnp.int32),
        ],
    )(ei_flat)


def _fused_body(nb, N, V, C,
                x_r, w_r, b_r, m_r, gamma_r, beta_r,
                out_r, a_s, dv_s, y_s, s1_r, s2_r, sc_r, sh_r):
    t = pl.program_id(0)
    j = pl.program_id(1)

    @pl.when(jnp.logical_and(t == 0, j == 0))
    def _():
        msub = m_r[:V, :V]
        deg = jnp.sum(msub, axis=1, keepdims=True)
        dinv = lax.rsqrt(deg)
        dv_s[...] = dinv
        a_s[...] = msub * dinv
        s1_r[...] = jnp.zeros_like(s1_r)
        s2_r[...] = jnp.zeros_like(s2_r)

    @pl.when(t == 0)
    def _():
        wv = w_r[...]
        bv = b_r[...]
        a = a_s[...]
        dinv = dv_s[...]
        s1 = jnp.zeros((V, C), jnp.float32)
        s2 = jnp.zeros((V, C), jnp.float32)
        base = j * nb
        for g in range(nb):
            xg = x_r[g]
            h = jnp.dot(xg, wv, preferred_element_type=jnp.float32) + bv
            agg = jnp.dot(a, dinv * h, preferred_element_type=jnp.float32)
            y = jnp.maximum(agg, 0.0)
            y_s[base + g] = y
            s1 = s1 + y
            s2 = s2 + y * y
        s1_r[...] = s1_r[...] + s1
        s2_r[...] = s2_r[...] + s2

    @pl.when(jnp.logical_and(t == 1, j == 0))
    def _():
        inv_n = 1.0 / N
        mean = s1_r[...] * inv_n
        var = s2_r[...] * inv_n - mean * mean
        rstd = lax.rsqrt(var + 1e-5)
        scale = rstd * gamma_r[...]
        sc_r[...] = scale
        sh_r[...] = beta_r[...] - mean * scale

    @pl.when(t == 1)
    def _():
        base = pl.multiple_of(j * nb, nb)
        yb = y_s[pl.ds(base, nb)]
        res = yb * sc_r[...][None, :, :] + sh_r[...][None, :, :]
        out_r[...] = res.reshape(nb, V * C)


def kernel(x, edge_index, W, b, gamma, beta):
    B_, T_, V, C = x.shape
    N = B_ * T_
    Co = W.shape[1]
    E = edge_index.shape[1]

    VR = ((V + 127) // 128) * 128
    EP = ((E + 1023) // 1024) * 1024

    ei = jnp.pad(edge_index.astype(jnp.int32), ((0, 0), (0, EP - E))).reshape(-1)

    m_flat = _sc_prep(ei, V, VR, E, EP)
    m = m_flat.reshape(VR, VR)

    x3 = x.reshape(N, V, C)
    b2 = b.reshape(1, Co)
    gamma2 = gamma.reshape(V, Co)
    beta2 = beta.reshape(V, Co)

    nb = 16
    NB = N // nb

    out = pl.pallas_call(
        functools.partial(_fused_body, nb, N, V, Co),
        grid=(2, NB),
        in_specs=[
            pl.BlockSpec((nb, V, C), lambda t, j: (j * (1 - t), 0, 0)),
            pl.BlockSpec((C, Co), lambda t, j: (0, 0)),
            pl.BlockSpec((1, Co), lambda t, j: (0, 0)),
            pl.BlockSpec((VR, VR), lambda t, j: (0, 0)),
            pl.BlockSpec((V, Co), lambda t, j: (0, 0)),
            pl.BlockSpec((V, Co), lambda t, j: (0, 0)),
        ],
        out_specs=pl.BlockSpec((nb, V * Co), lambda t, j: (j * t, 0)),
        out_shape=jax.ShapeDtypeStruct((N, V * Co), jnp.float32),
        scratch_shapes=[
            pltpu.VMEM((V, V), jnp.float32),
            pltpu.VMEM((V, 1), jnp.float32),
            pltpu.VMEM((N, V, Co), jnp.float32),
            pltpu.VMEM((V, Co), jnp.float32),
            pltpu.VMEM((V, Co), jnp.float32),
            pltpu.VMEM((V, Co), jnp.float32),
            pltpu.VMEM((V, Co), jnp.float32),
        ],
        compiler_params=pltpu.CompilerParams(
            dimension_semantics=("arbitrary", "arbitrary"),
        ),
    )(x3, W, b2, m, gamma2, beta2)

    return out.reshape(B_, T_, V * Co)


# speedup vs baseline: 1.2217x; 1.0105x over previous
import functools

import jax
import jax.numpy as jnp
from jax import lax
from jax.experimental import pallas as pl
from jax.experimental.pallas import tpu as pltpu
from jax.experimental.pallas import tpu_sc as plsc


def _sc_prep_body(V, VR, E, EP, RP,
                  ei_hbm, m_hbm,
                  slab, srcv, dstv):
    c = lax.axis_index("c")
    s = lax.axis_index("s")
    nc = 2
    wid = s * nc + c
    lo = wid * RP

    pltpu.sync_copy(ei_hbm.at[pl.ds(0, EP)], srcv)
    pltpu.sync_copy(ei_hbm.at[pl.ds(EP, EP)], dstv)

    zv16 = jnp.zeros((16,), jnp.float32)

    def zero_body(i, carry):
        slab[pl.ds(i * 16, 16)] = zv16
        return carry

    lax.fori_loop(0, (RP * VR) // 16, zero_body, 0)

    ones = jnp.ones((16,), jnp.float32)
    lane = lax.iota(jnp.int32, 16)

    def edge_body(i, carry):
        base = i * 16
        mask = (base + lane) < E
        sv = jnp.where(mask, srcv[pl.ds(base, 16)], 0)
        dv = jnp.where(mask, dstv[pl.ds(base, 16)], 0)
        r = dv - lo
        mask = jnp.logical_and(mask,
                               jnp.logical_and(r >= 0, r < RP))
        idx = r * VR + sv
        plsc.addupdate_scatter(slab, [idx], ones, mask=mask)
        return carry

    lax.fori_loop(0, (E + 15) // 16, edge_body, 0)

    v = lo + lane
    dmask = jnp.logical_and(lane < RP, v < V)
    didx = lane * (VR + 1) + lo
    plsc.addupdate_scatter(slab, [didx], ones, mask=dmask)

    pltpu.sync_copy(slab, m_hbm.at[pl.ds(lo * VR, RP * VR)])


def _sc_prep(ei_flat, V, VR, E, EP):
    mesh = plsc.VectorSubcoreMesh(core_axis_name="c", subcore_axis_name="s")
    RP = VR // 32
    body = functools.partial(_sc_prep_body, V, VR, E, EP, RP)
    return pl.kernel(
        body,
        out_type=jax.ShapeDtypeStruct((VR * VR,), jnp.float32),
        mesh=mesh,
        compiler_params=pltpu.CompilerParams(needs_layout_passes=False),
        scratch_types=[
            pltpu.VMEM((RP * VR,), jnp.float32),
            pltpu.VMEM((EP,), jnp.int32),
            pltpu.VMEM((EP,), jnp.int32),
        ],
    )(ei_flat)


def _fused_body(nb, N, V, C,
                x_r, w_r, b_r, m_r, gamma_r, beta_r,
                out_r, a_s, dv_s, y_s, s1_r, s2_r, sc_r, sh_r):
    t = pl.program_id(0)
    j = pl.program_id(1)

    @pl.when(jnp.logical_and(t == 0, j == 0))
    def _():
        msub = m_r[:V, :V]
        deg = jnp.sum(msub, axis=1, keepdims=True)
        dinv = lax.rsqrt(deg)
        dv_s[...] = dinv
        a_s[...] = msub * dinv
        s1_r[...] = jnp.zeros_like(s1_r)
        s2_r[...] = jnp.zeros_like(s2_r)

    @pl.when(t == 0)
    def _():
        wv = w_r[...]
        bv = b_r[...]
        a = a_s[...]
        dinv = dv_s[...]
        s1 = jnp.zeros((V, C), jnp.float32)
        s2 = jnp.zeros((V, C), jnp.float32)
        base = j * nb
        for g in range(nb):
            xg = x_r[g]
            h = jnp.dot(xg, wv, preferred_element_type=jnp.float32) + bv
            agg = jnp.dot(a, dinv * h, preferred_element_type=jnp.float32)
            y = jnp.maximum(agg, 0.0)
            y_s[base + g] = y
            s1 = s1 + y
            s2 = s2 + y * y
        s1_r[...] = s1_r[...] + s1
        s2_r[...] = s2_r[...] + s2

    @pl.when(jnp.logical_and(t == 1, j == 0))
    def _():
        inv_n = 1.0 / N
        mean = s1_r[...] * inv_n
        var = s2_r[...] * inv_n - mean * mean
        rstd = lax.rsqrt(var + 1e-5)
        scale = rstd * gamma_r[...]
        sc_r[...] = scale
        sh_r[...] = beta_r[...] - mean * scale

    @pl.when(t == 1)
    def _():
        base = pl.multiple_of(j * nb, nb)
        yb = y_s[pl.ds(base, nb)]
        res = yb * sc_r[...][None, :, :] + sh_r[...][None, :, :]
        out_r[...] = res.reshape(nb, V * C)


def kernel(x, edge_index, W, b, gamma, beta):
    B_, T_, V, C = x.shape
    N = B_ * T_
    Co = W.shape[1]
    E = edge_index.shape[1]

    VR = ((V + 127) // 128) * 128
    EP = ((E + 1023) // 1024) * 1024

    ei = jnp.pad(edge_index.astype(jnp.int32), ((0, 0), (0, EP - E))).reshape(-1)

    m_flat = _sc_prep(ei, V, VR, E, EP)
    m = m_flat.reshape(VR, VR)

    x3 = x.reshape(N, V, C)
    b2 = b.reshape(1, Co)
    gamma2 = gamma.reshape(V, Co)
    beta2 = beta.reshape(V, Co)

    nb = 32
    NB = N // nb

    out = pl.pallas_call(
        functools.partial(_fused_body, nb, N, V, Co),
        grid=(2, NB),
        in_specs=[
            pl.BlockSpec((nb, V, C), lambda t, j: (j * (1 - t), 0, 0)),
            pl.BlockSpec((C, Co), lambda t, j: (0, 0)),
            pl.BlockSpec((1, Co), lambda t, j: (0, 0)),
            pl.BlockSpec((VR, VR), lambda t, j: (0, 0)),
            pl.BlockSpec((V, Co), lambda t, j: (0, 0)),
            pl.BlockSpec((V, Co), lambda t, j: (0, 0)),
        ],
        out_specs=pl.BlockSpec((nb, V * Co), lambda t, j: (j * t, 0)),
        out_shape=jax.ShapeDtypeStruct((N, V * Co), jnp.float32),
        scratch_shapes=[
            pltpu.VMEM((V, V), jnp.float32),
            pltpu.VMEM((V, 1), jnp.float32),
            pltpu.VMEM((N, V, Co), jnp.float32),
            pltpu.VMEM((V, Co), jnp.float32),
            pltpu.VMEM((V, Co), jnp.float32),
            pltpu.VMEM((V, Co), jnp.float32),
            pltpu.VMEM((V, Co), jnp.float32),
        ],
        compiler_params=pltpu.CompilerParams(
            dimension_semantics=("arbitrary", "arbitrary"),
        ),
    )(x3, W, b2, m, gamma2, beta2)

    return out.reshape(B_, T_, V * Co)


# speedup vs baseline: 1.2506x; 1.0236x over previous
import functools

import jax
import jax.numpy as jnp
from jax import lax
from jax.experimental import pallas as pl
from jax.experimental.pallas import tpu as pltpu
from jax.experimental.pallas import tpu_sc as plsc


def _sc_prep_body(V, VR, E, EP, RP,
                  ei_hbm, m_hbm,
                  slab, srcv, dstv):
    c = lax.axis_index("c")
    s = lax.axis_index("s")
    nc = 2
    wid = s * nc + c
    lo = wid * RP

    pltpu.sync_copy(ei_hbm.at[pl.ds(0, EP)], srcv)
    pltpu.sync_copy(ei_hbm.at[pl.ds(EP, EP)], dstv)

    zv16 = jnp.zeros((16,), jnp.float32)

    def zero_body(i, carry):
        slab[pl.ds(i * 16, 16)] = zv16
        return carry

    lax.fori_loop(0, (RP * VR) // 16, zero_body, 0)

    ones = jnp.ones((16,), jnp.float32)
    lane = lax.iota(jnp.int32, 16)

    def edge_body(i, carry):
        base = i * 16
        mask = (base + lane) < E
        sv = jnp.where(mask, srcv[pl.ds(base, 16)], 0)
        dv = jnp.where(mask, dstv[pl.ds(base, 16)], 0)
        r = dv - lo
        mask = jnp.logical_and(mask,
                               jnp.logical_and(r >= 0, r < RP))
        idx = r * VR + sv
        plsc.addupdate_scatter(slab, [idx], ones, mask=mask)
        return carry

    lax.fori_loop(0, (E + 15) // 16, edge_body, 0)

    v = lo + lane
    dmask = jnp.logical_and(lane < RP, v < V)
    didx = lane * (VR + 1) + lo
    plsc.addupdate_scatter(slab, [didx], ones, mask=dmask)

    pltpu.sync_copy(slab, m_hbm.at[pl.ds(lo * VR, RP * VR)])


def _sc_prep(ei_flat, V, VR, E, EP):
    mesh = plsc.VectorSubcoreMesh(core_axis_name="c", subcore_axis_name="s")
    RP = VR // 32
    body = functools.partial(_sc_prep_body, V, VR, E, EP, RP)
    return pl.kernel(
        body,
        out_type=jax.ShapeDtypeStruct((VR * VR,), jnp.float32),
        mesh=mesh,
        compiler_params=pltpu.CompilerParams(needs_layout_passes=False),
        scratch_types=[
            pltpu.VMEM((RP * VR,), jnp.float32),
            pltpu.VMEM((EP,), jnp.int32),
            pltpu.VMEM((EP,), jnp.int32),
        ],
    )(ei_flat)


def _h_body(nb, V, C, x_r, w_r, b_r, h_r):
    wv = w_r[...]
    bv = b_r[...]
    for g in range(nb):
        h_r[g] = jnp.dot(x_r[g], wv, preferred_element_type=jnp.float32) + bv


def _fused_body(nb, N, V, C,
                h_r, m_r, gamma_r, beta_r,
                out_r, a_s, dv_s, y_s, s1_r, s2_r, sc_r, sh_r):
    t = pl.program_id(0)
    j = pl.program_id(1)

    @pl.when(jnp.logical_and(t == 0, j == 0))
    def _():
        msub = m_r[:V, :V]
        deg = jnp.sum(msub, axis=1, keepdims=True)
        dinv = lax.rsqrt(deg)
        dv_s[...] = dinv
        a_s[...] = msub * dinv
        s1_r[...] = jnp.zeros_like(s1_r)
        s2_r[...] = jnp.zeros_like(s2_r)

    @pl.when(t == 0)
    def _():
        a = a_s[...]
        dinv = dv_s[...]
        s1 = jnp.zeros((V, C), jnp.float32)
        s2 = jnp.zeros((V, C), jnp.float32)
        base = j * nb
        for g in range(nb):
            h = h_r[g]
            agg = jnp.dot(a, dinv * h, preferred_element_type=jnp.float32)
            y = jnp.maximum(agg, 0.0)
            y_s[base + g] = y
            s1 = s1 + y
            s2 = s2 + y * y
        s1_r[...] = s1_r[...] + s1
        s2_r[...] = s2_r[...] + s2

    @pl.when(jnp.logical_and(t == 1, j == 0))
    def _():
        inv_n = 1.0 / N
        mean = s1_r[...] * inv_n
        var = s2_r[...] * inv_n - mean * mean
        rstd = lax.rsqrt(var + 1e-5)
        scale = rstd * gamma_r[...]
        sc_r[...] = scale
        sh_r[...] = beta_r[...] - mean * scale

    @pl.when(t == 1)
    def _():
        base = pl.multiple_of(j * nb, nb)
        yb = y_s[pl.ds(base, nb)]
        res = yb * sc_r[...][None, :, :] + sh_r[...][None, :, :]
        out_r[...] = res.reshape(nb, V * C)


def kernel(x, edge_index, W, b, gamma, beta):
    B_, T_, V, C = x.shape
    N = B_ * T_
    Co = W.shape[1]
    E = edge_index.shape[1]

    VR = ((V + 127) // 128) * 128
    EP = ((E + 1023) // 1024) * 1024

    ei = jnp.pad(edge_index.astype(jnp.int32), ((0, 0), (0, EP - E))).reshape(-1)

    m_flat = _sc_prep(ei, V, VR, E, EP)
    m = m_flat.reshape(VR, VR)

    x3 = x.reshape(N, V, C)
    b2 = b.reshape(1, Co)
    gamma2 = gamma.reshape(V, Co)
    beta2 = beta.reshape(V, Co)

    nb = 32
    NB = N // nb

    h_all = pl.pallas_call(
        functools.partial(_h_body, nb, V, Co),
        grid=(NB,),
        in_specs=[
            pl.BlockSpec((nb, V, C), lambda j: (j, 0, 0)),
            pl.BlockSpec((C, Co), lambda j: (0, 0)),
            pl.BlockSpec((1, Co), lambda j: (0, 0)),
        ],
        out_specs=pl.BlockSpec((nb, V, Co), lambda j: (j, 0, 0)),
        out_shape=jax.ShapeDtypeStruct((N, V, Co), jnp.float32),
    )(x3, W, b2)

    out = pl.pallas_call(
        functools.partial(_fused_body, nb, N, V, Co),
        grid=(2, NB),
        in_specs=[
            pl.BlockSpec((nb, V, Co), lambda t, j: (j * (1 - t), 0, 0)),
            pl.BlockSpec((VR, VR), lambda t, j: (0, 0)),
            pl.BlockSpec((V, Co), lambda t, j: (0, 0)),
            pl.BlockSpec((V, Co), lambda t, j: (0, 0)),
        ],
        out_specs=pl.BlockSpec((nb, V * Co), lambda t, j: (j * t, 0)),
        out_shape=jax.ShapeDtypeStruct((N, V * Co), jnp.float32),
        scratch_shapes=[
            pltpu.VMEM((V, V), jnp.float32),
            pltpu.VMEM((V, 1), jnp.float32),
            pltpu.VMEM((N, V, Co), jnp.float32),
            pltpu.VMEM((V, Co), jnp.float32),
            pltpu.VMEM((V, Co), jnp.float32),
            pltpu.VMEM((V, Co), jnp.float32),
            pltpu.VMEM((V, Co), jnp.float32),
        ],
        compiler_params=pltpu.CompilerParams(
            dimension_semantics=("arbitrary", "arbitrary"),
        ),
    )(h_all, m, gamma2, beta2)

    return out.reshape(B_, T_, V * Co)
